# baseline (speedup 1.0000x reference)
import jax
import jax.numpy as jnp
from jax.experimental import pallas as pl

ROWS = 16384
COLS = 2048
BLOCK_ROWS = 1024
LANES = 128
NUM_SUB = COLS // LANES


def _flip_block(in_ref, out_ref):
    rev = (LANES - 1) - jax.lax.broadcasted_iota(
        jnp.int32, (BLOCK_ROWS, LANES), 1
    )
    for j in range(NUM_SUB):
        src = NUM_SUB - 1 - j
        x = in_ref[:, src * LANES:(src + 1) * LANES]
        out_ref[:, j * LANES:(j + 1) * LANES] = jnp.take_along_axis(
            x, rev, axis=1
        )


def kernel(inputs, cond_inputs):
    out = pl.pallas_call(
        _flip_block,
        grid=(ROWS // BLOCK_ROWS,),
        in_specs=[pl.BlockSpec((BLOCK_ROWS, COLS), lambda i: (i, 0))],
        out_specs=pl.BlockSpec((BLOCK_ROWS, COLS), lambda i: (i, 0)),
        out_shape=jax.ShapeDtypeStruct((ROWS, COLS), inputs.dtype),
    )(inputs)
    return (out, 0.0)
